# baseline (speedup 1.0000x reference)
import functools

import jax
import jax.numpy as jnp
from jax import lax
from jax.experimental import pallas as pl
from jax.experimental.pallas import tpu as pltpu
from jax.experimental.pallas import tpu_sc as plsc

D = 64
NC = 2
NS = 16
NW = NC * NS
CHUNK = 512


def _gather_body(table_hbm, idx_hbm, out_hbm, idx_v, rows_v, sem, *, b_per_w, n_chunk):
    wid = lax.axis_index("s") * NC + lax.axis_index("c")
    base = wid * b_per_w

    def chunk(c, carry):
        off = base + c * CHUNK
        pltpu.sync_copy(idx_hbm.at[pl.ds(off, CHUNK)], idx_v)
        pltpu.async_copy(table_hbm.at[idx_v], rows_v, sem).wait()
        pltpu.sync_copy(rows_v, out_hbm.at[pl.ds(off, CHUNK)])
        return carry

    lax.fori_loop(0, n_chunk, chunk, 0)


@functools.partial(jax.jit, static_argnames=("b_total",))
def _gather(table, idx_flat, b_total):
    b_per_w = b_total // NW
    n_chunk = b_per_w // CHUNK
    mesh = plsc.VectorSubcoreMesh(core_axis_name="c", subcore_axis_name="s")
    body = functools.partial(_gather_body, b_per_w=b_per_w, n_chunk=n_chunk)
    return pl.kernel(
        body,
        out_type=jax.ShapeDtypeStruct((b_total, D), jnp.float32),
        mesh=mesh,
        scratch_types=[
            pltpu.VMEM((CHUNK,), jnp.int32),
            pltpu.VMEM((CHUNK, D), jnp.float32),
            pltpu.SemaphoreType.DMA,
        ],
        compiler_params=pltpu.CompilerParams(use_tc_tiling_on_sc=False),
    )(table, idx_flat)


def kernel(x, table):
    b, h = x.shape
    idx_flat = x.reshape(-1).astype(jnp.int32)
    out = _gather(table, idx_flat, b * h)
    return out.reshape(b, h, D)


# speedup vs baseline: 1.0462x; 1.0462x over previous
import functools

import jax
import jax.numpy as jnp
from jax import lax
from jax.experimental import pallas as pl
from jax.experimental.pallas import tpu as pltpu
from jax.experimental.pallas import tpu_sc as plsc

D = 64
NC = 2
NS = 16
NW = NC * NS
CHUNK = 640


def _gather_body(table_hbm, idx_hbm, out_hbm, idx_v, rows_v,
                 sem_g0, sem_g1, sem_g2, sem_s0, sem_s1, sem_s2,
                 *, b_per_w, n_chunk):
    wid = lax.axis_index("s") * NC + lax.axis_index("c")
    base = wid * b_per_w
    sem_g = (sem_g0, sem_g1, sem_g2)
    sem_s = (sem_s0, sem_s1, sem_s2)

    def load_idx(c, b):
        pltpu.sync_copy(idx_hbm.at[pl.ds(base + c * CHUNK, CHUNK)], idx_v.at[b])

    def gather_cp(b):
        return pltpu.make_async_copy(table_hbm.at[idx_v.at[b]], rows_v.at[b], sem_g[b])

    def store_cp(c, b):
        return pltpu.make_async_copy(
            rows_v.at[b], out_hbm.at[pl.ds(base + c * CHUNK, CHUNK)], sem_s[b])

    def step(c, A, C):
        gather_cp(A).wait()
        store_cp(c, A).start()
        store_cp(c - 1, C).wait()
        load_idx(c + 2, C)
        gather_cp(C).start()

    for b in (0, 1):
        load_idx(b, b)
        gather_cp(b).start()
    gather_cp(0).wait()
    store_cp(0, 0).start()
    load_idx(2, 2)
    gather_cp(2).start()
    step(1, 1, 0)

    def loop_body(g, carry):
        c0 = 2 + 3 * g
        step(c0, 2, 1)
        step(c0 + 1, 0, 2)
        step(c0 + 2, 1, 0)
        return carry

    lax.fori_loop(0, (n_chunk - 4) // 3, loop_body, 0)

    nA = (n_chunk - 2) % 3
    nC = (n_chunk - 3) % 3
    gather_cp(nA).wait()
    store_cp(n_chunk - 2, nA).start()
    store_cp(n_chunk - 3, nC).wait()
    lA = (n_chunk - 1) % 3
    gather_cp(lA).wait()
    store_cp(n_chunk - 1, lA).start()
    store_cp(n_chunk - 2, nA).wait()
    store_cp(n_chunk - 1, lA).wait()


@functools.partial(jax.jit, static_argnames=("b_total",))
def _gather(table, idx_flat, b_total):
    b_per_w = b_total // NW
    n_chunk = b_per_w // CHUNK
    assert n_chunk >= 4 and (n_chunk - 4) % 3 == 0
    mesh = plsc.VectorSubcoreMesh(core_axis_name="c", subcore_axis_name="s")
    body = functools.partial(_gather_body, b_per_w=b_per_w, n_chunk=n_chunk)
    return pl.kernel(
        body,
        out_type=jax.ShapeDtypeStruct((b_total, D), jnp.float32),
        mesh=mesh,
        scratch_types=[
            pltpu.VMEM((3, CHUNK), jnp.int32),
            pltpu.VMEM((3, CHUNK, D), jnp.float32),
            pltpu.SemaphoreType.DMA,
            pltpu.SemaphoreType.DMA,
            pltpu.SemaphoreType.DMA,
            pltpu.SemaphoreType.DMA,
            pltpu.SemaphoreType.DMA,
            pltpu.SemaphoreType.DMA,
        ],
        compiler_params=pltpu.CompilerParams(use_tc_tiling_on_sc=False),
    )(table, idx_flat)


def kernel(x, table):
    b, h = x.shape
    idx_flat = x.reshape(-1).astype(jnp.int32)
    out = _gather(table, idx_flat, b * h)
    return out.reshape(b, h, D)
